# baseline (speedup 1.0000x reference)
import jax
import jax.numpy as jnp
from jax import lax
from jax.experimental import pallas as pl
from jax.experimental.pallas import tpu as pltpu
from jax.experimental.pallas import tpu_sc as plsc

N_CAT = 26
N_NUM = 13
D = 32
B = 16384
CAT_SIZE = 100000

NC = 2
NS = 16
NW = NC * NS
BPW = B // NW
IDX_CHUNK = 128
SPF = BPW // IDX_CHUNK
L = 16


def _body(xnum_hbm, xcat_hbm, nw_hbm, nb_hbm, tbl_hbm, cb_hbm, out_hbm,
          xcat_v, xnum_v, idxc_v, gbuf0, gbuf1, obuf0, obuf1,
          nw_v, nb_v, cb_v, gsem0, gsem1, osem0, osem1):
    wid = lax.axis_index("c") * NS + lax.axis_index("s")
    b0 = wid * BPW

    pltpu.sync_copy(xcat_hbm.at[pl.ds(b0, BPW), :], xcat_v)
    pltpu.sync_copy(xnum_hbm.at[pl.ds(b0, BPW), :], xnum_v)
    pltpu.sync_copy(nw_hbm, nw_v)
    pltpu.sync_copy(nb_hbm, nb_v)
    pltpu.sync_copy(cb_hbm, cb_v)

    lanes = lax.iota(jnp.int32, L)

    def build_col(f):
        col = jnp.full((L,), f, jnp.int32)
        off = jnp.full((L,), f * CAT_SIZE, jnp.int32)

        @pl.loop(0, BPW // L, unroll=4)
        def _(k, f=f, col=col, off=off):
            rows = k * L + lanes
            vals = plsc.load_gather(xcat_v, [rows, col])
            idxc_v[f, pl.ds(k * L, L)] = vals + off

    gbuf = (gbuf0, gbuf1)
    obuf = (obuf0, obuf1)
    gsems = (gsem0, gsem1)
    osems = (osem0, osem1)

    def fire_gather(f):
        descs = []
        for c in range(SPF):
            descs.append(pltpu.async_copy(
                tbl_hbm.at[idxc_v.at[f, pl.ds(c * IDX_CHUNK, IDX_CHUNK)]],
                gbuf[f % 2].at[pl.ds(c * IDX_CHUNK, IDX_CHUNK), :],
                gsems[f % 2]))
        return descs

    build_col(0)
    g_descs = {0: fire_gather(0)}

    out_descs = {}
    for f in range(N_CAT):
        buf = f % 2
        if f + 1 < N_CAT:
            build_col(f + 1)
            g_descs[f + 1] = fire_gather(f + 1)
        for d in g_descs[f]:
            d.wait()
        if f - 2 >= 0:
            out_descs[f - 2].wait()
        g = gbuf[buf]
        o = obuf[buf]
        cb_lo = cb_v[f, pl.ds(0, L)]
        cb_hi = cb_v[f, pl.ds(L, L)]

        @pl.loop(0, BPW, unroll=8)
        def _bias_add(i, g=g, o=o, cb_lo=cb_lo, cb_hi=cb_hi):
            o[i, 0, pl.ds(0, L)] = g[i, pl.ds(0, L)] + cb_lo
            o[i, 0, pl.ds(L, L)] = g[i, pl.ds(L, L)] + cb_hi

        out_descs[f] = pltpu.async_copy(
            o, out_hbm.at[pl.ds(b0, BPW), pl.ds(N_NUM + f, 1), :],
            osems[buf])

    num_descs = {}
    for j in range(N_NUM):
        buf = j % 2
        if j < 2:
            out_descs[N_CAT - 2 + j].wait()
        else:
            num_descs[j - 2].wait()
        o = obuf[buf]
        colj = jnp.full((L,), j, jnp.int32)
        w_lo = nw_v[j, pl.ds(0, L)]
        w_hi = nw_v[j, pl.ds(L, L)]
        a_lo = nb_v[j, pl.ds(0, L)]
        a_hi = nb_v[j, pl.ds(L, L)]

        @pl.loop(0, BPW // L)
        def _num_emb(t, o=o, colj=colj, w_lo=w_lo, w_hi=w_hi,
                     a_lo=a_lo, a_hi=a_hi):
            rows = t * L + lanes
            xv = plsc.load_gather(xnum_v, [rows, colj])
            for e in range(L):
                xs = xv[e]
                o[t * L + e, 0, pl.ds(0, L)] = w_lo * xs + a_lo
                o[t * L + e, 0, pl.ds(L, L)] = w_hi * xs + a_hi

        num_descs[j] = pltpu.async_copy(
            o, out_hbm.at[pl.ds(b0, BPW), pl.ds(j, 1), :], osems[buf])

    num_descs[N_NUM - 2].wait()
    num_descs[N_NUM - 1].wait()


_sc_tokenize = pl.kernel(
    _body,
    out_type=jax.ShapeDtypeStruct((B, N_NUM + N_CAT, D), jnp.float32),
    mesh=plsc.VectorSubcoreMesh(core_axis_name="c", subcore_axis_name="s",
                                num_cores=NC, num_subcores=NS),
    compiler_params=pltpu.CompilerParams(use_tc_tiling_on_sc=False,
                                         needs_layout_passes=False),
    scratch_types=[
        pltpu.VMEM((BPW, N_CAT), jnp.int32),
        pltpu.VMEM((BPW, N_NUM), jnp.float32),
        pltpu.VMEM((N_CAT, BPW), jnp.int32),
        pltpu.VMEM((BPW, D), jnp.float32),
        pltpu.VMEM((BPW, D), jnp.float32),
        pltpu.VMEM((BPW, 1, D), jnp.float32),
        pltpu.VMEM((BPW, 1, D), jnp.float32),
        pltpu.VMEM((N_NUM, D), jnp.float32),
        pltpu.VMEM((N_NUM, D), jnp.float32),
        pltpu.VMEM((N_CAT, D), jnp.float32),
        pltpu.SemaphoreType.DMA,
        pltpu.SemaphoreType.DMA,
        pltpu.SemaphoreType.DMA,
        pltpu.SemaphoreType.DMA,
    ],
)


_TCOLS = 512


def _relayout_body(src_ref, dst_ref):
    x = src_ref[...]
    xt = jnp.transpose(x).reshape(_TCOLS // 4, 4, 32)
    for q in range(4):
        dst_ref[:, q * 32:(q + 1) * 32] = xt[:, q, :]


_tbl_relayout = pl.pallas_call(
    _relayout_body,
    grid=((2600000 + _TCOLS - 1) // _TCOLS,),
    in_specs=[pl.BlockSpec((32, _TCOLS), lambda m: (0, m))],
    out_specs=pl.BlockSpec((_TCOLS // 4, 128), lambda m: (m, 0)),
    out_shape=jax.ShapeDtypeStruct((650000, 128), jnp.float32),
)


@jax.jit
def kernel(x_num, x_cat, num_weight, num_bias, cat_table, cat_bias):
    tbl_t = jax.lax.optimization_barrier(cat_table.T)
    tbl_lin = jax.lax.optimization_barrier(_tbl_relayout(tbl_t))
    tbl2 = jnp.reshape(tbl_lin, (2600000, 32))
    return _sc_tokenize(jnp.asarray(x_num, jnp.float32),
                        jnp.asarray(x_cat, jnp.int32),
                        num_weight, num_bias, tbl2, cat_bias)


# speedup vs baseline: 2.5072x; 2.5072x over previous
import jax
import jax.numpy as jnp
from jax import lax
from jax.experimental import pallas as pl
from jax.experimental.pallas import tpu as pltpu
from jax.experimental.pallas import tpu_sc as plsc

N_CAT = 26
N_NUM = 13
D = 32
B = 16384
CAT_SIZE = 100000

NC = 2
NS = 16
NW = NC * NS
BPW = B // NW
IDX_CHUNK = 128
SPF = BPW // IDX_CHUNK
L = 16


def _body(xnum_hbm, xcat_hbm, nw_hbm, nb_hbm, tbl_hbm, cb_hbm, out_hbm,
          xcat_v, xnum_v, idxc_v, gbuf0, gbuf1, obuf0, obuf1,
          nw_v, nb_v, cb_v, gsem0, gsem1, osem0, osem1):
    wid = lax.axis_index("c") * NS + lax.axis_index("s")
    b0 = wid * BPW

    pltpu.sync_copy(xcat_hbm.at[pl.ds(b0, BPW), :], xcat_v)
    pltpu.sync_copy(xnum_hbm.at[pl.ds(b0, BPW), :], xnum_v)
    pltpu.sync_copy(nw_hbm, nw_v)
    pltpu.sync_copy(nb_hbm, nb_v)
    pltpu.sync_copy(cb_hbm, cb_v)

    lanes = lax.iota(jnp.int32, L)

    def build_col(f):
        col = jnp.full((L,), f, jnp.int32)
        off = jnp.full((L,), f * CAT_SIZE, jnp.int32)

        @pl.loop(0, BPW // L, unroll=4)
        def _(k, f=f, col=col, off=off):
            rows = k * L + lanes
            vals = plsc.load_gather(xcat_v, [rows, col])
            idxc_v[f, pl.ds(k * L, L)] = vals + off

    gbuf = (gbuf0, gbuf1)
    obuf = (obuf0, obuf1)
    gsems = (gsem0, gsem1)
    osems = (osem0, osem1)

    def fire_gather(f):
        descs = []
        for c in range(SPF):
            descs.append(pltpu.async_copy(
                tbl_hbm.at[idxc_v.at[f, pl.ds(c * IDX_CHUNK, IDX_CHUNK)]],
                gbuf[f % 2].at[pl.ds(c * IDX_CHUNK, IDX_CHUNK), :],
                gsems[f % 2]))
        return descs

    build_col(0)
    g_descs = {0: fire_gather(0)}

    out_descs = {}
    for f in range(N_CAT):
        buf = f % 2
        if f + 1 < N_CAT:
            build_col(f + 1)
            g_descs[f + 1] = fire_gather(f + 1)
        for d in g_descs[f]:
            d.wait()
        if f - 2 >= 0:
            out_descs[f - 2].wait()
        g = gbuf[buf]
        o = obuf[buf]
        cb_lo = cb_v[f, pl.ds(0, L)]
        cb_hi = cb_v[f, pl.ds(L, L)]

        @pl.loop(0, BPW, unroll=8)
        def _bias_add(i, g=g, o=o, cb_lo=cb_lo, cb_hi=cb_hi):
            o[i, 0, pl.ds(0, L)] = g[i, pl.ds(0, L)] + cb_lo
            o[i, 0, pl.ds(L, L)] = g[i, pl.ds(L, L)] + cb_hi

        out_descs[f] = pltpu.async_copy(
            o, out_hbm.at[pl.ds(b0, BPW), pl.ds(N_NUM + f, 1), :],
            osems[buf])

    num_descs = {}
    for j in range(N_NUM):
        buf = j % 2
        if j < 2:
            out_descs[N_CAT - 2 + j].wait()
        else:
            num_descs[j - 2].wait()
        o = obuf[buf]
        colj = jnp.full((L,), j, jnp.int32)
        w_lo = nw_v[j, pl.ds(0, L)]
        w_hi = nw_v[j, pl.ds(L, L)]
        a_lo = nb_v[j, pl.ds(0, L)]
        a_hi = nb_v[j, pl.ds(L, L)]

        @pl.loop(0, BPW // L)
        def _num_emb(t, o=o, colj=colj, w_lo=w_lo, w_hi=w_hi,
                     a_lo=a_lo, a_hi=a_hi):
            rows = t * L + lanes
            xv = plsc.load_gather(xnum_v, [rows, colj])
            for e in range(L):
                xs = xv[e]
                o[t * L + e, 0, pl.ds(0, L)] = w_lo * xs + a_lo
                o[t * L + e, 0, pl.ds(L, L)] = w_hi * xs + a_hi

        num_descs[j] = pltpu.async_copy(
            o, out_hbm.at[pl.ds(b0, BPW), pl.ds(j, 1), :], osems[buf])

    num_descs[N_NUM - 2].wait()
    num_descs[N_NUM - 1].wait()


_sc_tokenize = pl.kernel(
    _body,
    out_type=jax.ShapeDtypeStruct((B, N_NUM + N_CAT, D), jnp.float32),
    mesh=plsc.VectorSubcoreMesh(core_axis_name="c", subcore_axis_name="s",
                                num_cores=NC, num_subcores=NS),
    compiler_params=pltpu.CompilerParams(use_tc_tiling_on_sc=False,
                                         needs_layout_passes=False),
    scratch_types=[
        pltpu.VMEM((BPW, N_CAT), jnp.int32),
        pltpu.VMEM((BPW, N_NUM), jnp.float32),
        pltpu.VMEM((N_CAT, BPW), jnp.int32),
        pltpu.VMEM((BPW, D), jnp.float32),
        pltpu.VMEM((BPW, D), jnp.float32),
        pltpu.VMEM((BPW, 1, D), jnp.float32),
        pltpu.VMEM((BPW, 1, D), jnp.float32),
        pltpu.VMEM((N_NUM, D), jnp.float32),
        pltpu.VMEM((N_NUM, D), jnp.float32),
        pltpu.VMEM((N_CAT, D), jnp.float32),
        pltpu.SemaphoreType.DMA,
        pltpu.SemaphoreType.DMA,
        pltpu.SemaphoreType.DMA,
        pltpu.SemaphoreType.DMA,
    ],
)

_TCOLS = 8192


def _relayout_body(src_ref, dst_ref):
    x = src_ref[...]
    xt = jnp.transpose(x).reshape(_TCOLS // 4, 4, 32)
    for q in range(4):
        dst_ref[:, q * 32:(q + 1) * 32] = xt[:, q, :]


_tbl_relayout = pl.pallas_call(
    _relayout_body,
    grid=((2600000 + _TCOLS - 1) // _TCOLS,),
    in_specs=[pl.BlockSpec((32, _TCOLS), lambda m: (0, m))],
    out_specs=pl.BlockSpec((_TCOLS // 4, 128), lambda m: (m, 0)),
    out_shape=jax.ShapeDtypeStruct((650000, 128), jnp.float32),
)


@jax.jit
def kernel(x_num, x_cat, num_weight, num_bias, cat_table, cat_bias):
    tbl_t = jax.lax.optimization_barrier(cat_table.T)
    tbl_lin = jax.lax.optimization_barrier(_tbl_relayout(tbl_t))
    tbl2 = jnp.reshape(tbl_lin, (2600000, 32))
    return _sc_tokenize(jnp.asarray(x_num, jnp.float32),
                        jnp.asarray(x_cat, jnp.int32),
                        num_weight, num_bias, tbl2, cat_bias)


# speedup vs baseline: 2.6876x; 1.0719x over previous
import jax
import jax.numpy as jnp
from jax import lax
from jax.experimental import pallas as pl
from jax.experimental.pallas import tpu as pltpu
from jax.experimental.pallas import tpu_sc as plsc

N_CAT = 26
N_NUM = 13
D = 32
B = 16384
CAT_SIZE = 100000

NC = 2
NS = 16
NW = NC * NS
BPW = B // NW
IDX_CHUNK = 128
SPF = BPW // IDX_CHUNK
L = 16


def _body(xnum_hbm, xcat_hbm, nw_hbm, nb_hbm, tbl_hbm, cb_hbm, out_hbm,
          xcat_v, xnum_v, idxc_v, gbuf0, gbuf1, obuf0, obuf1,
          nw_v, nb_v, cb_v, gsem0, gsem1, osem0, osem1):
    wid = lax.axis_index("c") * NS + lax.axis_index("s")
    b0 = wid * BPW

    pltpu.sync_copy(xcat_hbm.at[pl.ds(b0, BPW), :], xcat_v)
    pltpu.sync_copy(xnum_hbm.at[pl.ds(b0, BPW), :], xnum_v)
    pltpu.sync_copy(nw_hbm, nw_v)
    pltpu.sync_copy(nb_hbm, nb_v)
    pltpu.sync_copy(cb_hbm, cb_v)

    lanes = lax.iota(jnp.int32, L)

    def build_col(f):
        col = jnp.full((L,), f, jnp.int32)
        off = jnp.full((L,), f * CAT_SIZE, jnp.int32)

        @pl.loop(0, BPW // L, unroll=4)
        def _(k, f=f, col=col, off=off):
            rows = k * L + lanes
            vals = plsc.load_gather(xcat_v, [rows, col])
            idxc_v[f, pl.ds(k * L, L)] = vals + off

    gbuf = (gbuf0, gbuf1)
    obuf = (obuf0, obuf1)
    gsems = (gsem0, gsem1)
    osems = (osem0, osem1)

    def fire_gather(f):
        descs = []
        for c in range(SPF):
            descs.append(pltpu.async_copy(
                tbl_hbm.at[idxc_v.at[f, pl.ds(c * IDX_CHUNK, IDX_CHUNK)]],
                gbuf[f % 2].at[pl.ds(c * IDX_CHUNK, IDX_CHUNK), :],
                gsems[f % 2]))
        return descs

    build_col(0)
    g_descs = {0: fire_gather(0)}

    out_descs = {}
    for f in range(N_CAT):
        buf = f % 2
        if f + 1 < N_CAT:
            build_col(f + 1)
            g_descs[f + 1] = fire_gather(f + 1)
        for d in g_descs[f]:
            d.wait()
        if f - 2 >= 0:
            out_descs[f - 2].wait()
        g = gbuf[buf]
        o = obuf[buf]
        cb_lo = cb_v[f, pl.ds(0, L)]
        cb_hi = cb_v[f, pl.ds(L, L)]

        @pl.loop(0, BPW // L)
        def _bias_tr(i, g=g, o=o, cb_lo=cb_lo, cb_hi=cb_hi):
            ml = i // (IDX_CHUNK // L)
            gg = i % (IDX_CHUNK // L)
            rows = i * L + lanes
            for d in range(D):
                bias_d = cb_lo[d] if d < L else cb_hi[d - L]
                colv = jnp.full((L,), d, jnp.int32)
                vals = plsc.load_gather(g, [rows, colv]) + bias_d
                o[d // 8, ml, d % 8, pl.ds(gg * L, L)] = vals

        out_descs[f] = pltpu.async_copy(
            o, out_hbm.at[N_NUM + f, :, pl.ds(4 * wid, SPF), :, :],
            osems[buf])

    num_descs = {}
    for j in range(N_NUM):
        buf = j % 2
        if j < 2:
            out_descs[N_CAT - 2 + j].wait()
        else:
            num_descs[j - 2].wait()
        o = obuf[buf]
        colj = jnp.full((L,), j, jnp.int32)
        w_lo = nw_v[j, pl.ds(0, L)]
        w_hi = nw_v[j, pl.ds(L, L)]
        a_lo = nb_v[j, pl.ds(0, L)]
        a_hi = nb_v[j, pl.ds(L, L)]

        @pl.loop(0, BPW // L)
        def _num_emb(i, o=o, colj=colj, w_lo=w_lo, w_hi=w_hi,
                     a_lo=a_lo, a_hi=a_hi):
            ml = i // (IDX_CHUNK // L)
            gg = i % (IDX_CHUNK // L)
            rows = i * L + lanes
            xv = plsc.load_gather(xnum_v, [rows, colj])
            for d in range(D):
                w_d = w_lo[d] if d < L else w_hi[d - L]
                a_d = a_lo[d] if d < L else a_hi[d - L]
                o[d // 8, ml, d % 8, pl.ds(gg * L, L)] = xv * w_d + a_d

        num_descs[j] = pltpu.async_copy(
            o, out_hbm.at[j, :, pl.ds(4 * wid, SPF), :, :], osems[buf])

    num_descs[N_NUM - 2].wait()
    num_descs[N_NUM - 1].wait()


_sc_tokenize = pl.kernel(
    _body,
    out_type=jax.ShapeDtypeStruct((N_NUM + N_CAT, D // 8, B // 128, 8, 128),
                                  jnp.float32),
    mesh=plsc.VectorSubcoreMesh(core_axis_name="c", subcore_axis_name="s",
                                num_cores=NC, num_subcores=NS),
    compiler_params=pltpu.CompilerParams(use_tc_tiling_on_sc=False,
                                         needs_layout_passes=False),
    scratch_types=[
        pltpu.VMEM((BPW, N_CAT), jnp.int32),
        pltpu.VMEM((BPW, N_NUM), jnp.float32),
        pltpu.VMEM((N_CAT, BPW), jnp.int32),
        pltpu.VMEM((BPW, D), jnp.float32),
        pltpu.VMEM((BPW, D), jnp.float32),
        pltpu.VMEM((D // 8, SPF, 8, 128), jnp.float32),
        pltpu.VMEM((D // 8, SPF, 8, 128), jnp.float32),
        pltpu.VMEM((N_NUM, D), jnp.float32),
        pltpu.VMEM((N_NUM, D), jnp.float32),
        pltpu.VMEM((N_CAT, D), jnp.float32),
        pltpu.SemaphoreType.DMA,
        pltpu.SemaphoreType.DMA,
        pltpu.SemaphoreType.DMA,
        pltpu.SemaphoreType.DMA,
    ],
)

_TCOLS = 8192


def _relayout_body(src_ref, dst_ref):
    x = src_ref[...]
    xt = jnp.transpose(x).reshape(_TCOLS // 4, 4, 32)
    for q in range(4):
        dst_ref[:, q * 32:(q + 1) * 32] = xt[:, q, :]


_tbl_relayout = pl.pallas_call(
    _relayout_body,
    grid=((2600000 + _TCOLS - 1) // _TCOLS,),
    in_specs=[pl.BlockSpec((32, _TCOLS), lambda m: (0, m))],
    out_specs=pl.BlockSpec((_TCOLS // 4, 128), lambda m: (m, 0)),
    out_shape=jax.ShapeDtypeStruct((650000, 128), jnp.float32),
)


@jax.jit
def kernel(x_num, x_cat, num_weight, num_bias, cat_table, cat_bias):
    tbl_t = jax.lax.optimization_barrier(cat_table.T)
    tbl_lin = jax.lax.optimization_barrier(_tbl_relayout(tbl_t))
    tbl2 = jnp.reshape(tbl_lin, (2600000, 32))
    out5 = _sc_tokenize(jnp.asarray(x_num, jnp.float32),
                        jnp.asarray(x_cat, jnp.int32),
                        num_weight, num_bias, tbl2, cat_bias)
    return out5.transpose((2, 4, 0, 1, 3)).reshape(B, N_NUM + N_CAT, D)


# speedup vs baseline: 4.5811x; 1.7046x over previous
import jax
import jax.numpy as jnp
from jax import lax
from jax.experimental import pallas as pl
from jax.experimental.pallas import tpu as pltpu
from jax.experimental.pallas import tpu_sc as plsc

N_CAT = 26
N_NUM = 13
D = 32
B = 16384
CAT_SIZE = 100000

NC = 2
NS = 16
NW = NC * NS
BPW = B // NW
IDX_CHUNK = 128
SPF = BPW // IDX_CHUNK
L = 16
QK = 655360


def _body(xnum_hbm, xcat_hbm, nw_hbm, nb_hbm, tbl_hbm, cb_hbm, out_hbm,
          xcat_v, xnum_v, idxc_v, gbuf0, gbuf1, obuf0, obuf1,
          nw_v, nb_v, cb_v, gsem0, gsem1, osem0, osem1):
    wid = lax.axis_index("c") * NS + lax.axis_index("s")
    b0 = wid * BPW

    pltpu.sync_copy(xcat_hbm.at[pl.ds(b0, BPW), :], xcat_v)
    pltpu.sync_copy(xnum_hbm.at[pl.ds(b0, BPW), :], xnum_v)
    pltpu.sync_copy(nw_hbm, nw_v)
    pltpu.sync_copy(nb_hbm, nb_v)
    pltpu.sync_copy(cb_hbm, cb_v)

    lanes = lax.iota(jnp.int32, L)

    def build_col(f):
        col = jnp.full((L,), f, jnp.int32)
        off = jnp.full((L,), f * CAT_SIZE, jnp.int32)

        @pl.loop(0, BPW // L)
        def _(k, f=f, col=col, off=off):
            rows = k * L + lanes
            vals = plsc.load_gather(xcat_v, [rows, col]) + off
            q = ((vals >= QK).astype(jnp.int32)
                 + (vals >= 2 * QK).astype(jnp.int32)
                 + (vals >= 3 * QK).astype(jnp.int32))
            idxc_v[f, pl.ds(k * L, L)] = 4 * (vals - q * QK) + q

    gbuf = (gbuf0, gbuf1)
    obuf = (obuf0, obuf1)
    gsems = (gsem0, gsem1)
    osems = (osem0, osem1)

    def fire_gather(f):
        descs = []
        for c in range(SPF):
            descs.append(pltpu.async_copy(
                tbl_hbm.at[idxc_v.at[f, pl.ds(c * IDX_CHUNK, IDX_CHUNK)]],
                gbuf[f % 2].at[pl.ds(c * IDX_CHUNK, IDX_CHUNK), :],
                gsems[f % 2]))
        return descs

    build_col(0)
    g_descs = {0: fire_gather(0)}

    out_descs = {}
    for f in range(N_CAT):
        buf = f % 2
        if f + 1 < N_CAT:
            build_col(f + 1)
            g_descs[f + 1] = fire_gather(f + 1)
        for d in g_descs[f]:
            d.wait()
        if f - 2 >= 0:
            out_descs[f - 2].wait()
        g = gbuf[buf]
        o = obuf[buf]
        cb_lo = cb_v[f, pl.ds(0, L)]
        cb_hi = cb_v[f, pl.ds(L, L)]

        @pl.loop(0, BPW // L)
        def _bias_tr(i, g=g, o=o, cb_lo=cb_lo, cb_hi=cb_hi):
            ml = i // (IDX_CHUNK // L)
            gg = i % (IDX_CHUNK // L)
            rows = i * L + lanes
            for d in range(D):
                bias_d = cb_lo[d] if d < L else cb_hi[d - L]
                colv = jnp.full((L,), d, jnp.int32)
                vals = plsc.load_gather(g, [rows, colv]) + bias_d
                o[d // 8, ml, d % 8, pl.ds(gg * L, L)] = vals

        out_descs[f] = pltpu.async_copy(
            o, out_hbm.at[N_NUM + f, :, pl.ds(4 * wid, SPF), :, :],
            osems[buf])

    num_descs = {}
    for j in range(N_NUM):
        buf = j % 2
        if j < 2:
            out_descs[N_CAT - 2 + j].wait()
        else:
            num_descs[j - 2].wait()
        o = obuf[buf]
        colj = jnp.full((L,), j, jnp.int32)
        w_lo = nw_v[j, pl.ds(0, L)]
        w_hi = nw_v[j, pl.ds(L, L)]
        a_lo = nb_v[j, pl.ds(0, L)]
        a_hi = nb_v[j, pl.ds(L, L)]

        @pl.loop(0, BPW // L)
        def _num_emb(i, o=o, colj=colj, w_lo=w_lo, w_hi=w_hi,
                     a_lo=a_lo, a_hi=a_hi):
            ml = i // (IDX_CHUNK // L)
            gg = i % (IDX_CHUNK // L)
            rows = i * L + lanes
            xv = plsc.load_gather(xnum_v, [rows, colj])
            for d in range(D):
                w_d = w_lo[d] if d < L else w_hi[d - L]
                a_d = a_lo[d] if d < L else a_hi[d - L]
                o[d // 8, ml, d % 8, pl.ds(gg * L, L)] = xv * w_d + a_d

        num_descs[j] = pltpu.async_copy(
            o, out_hbm.at[j, :, pl.ds(4 * wid, SPF), :, :], osems[buf])

    num_descs[N_NUM - 2].wait()
    num_descs[N_NUM - 1].wait()


_sc_tokenize = pl.kernel(
    _body,
    out_type=jax.ShapeDtypeStruct((N_NUM + N_CAT, D // 8, B // 128, 8, 128),
                                  jnp.float32),
    mesh=plsc.VectorSubcoreMesh(core_axis_name="c", subcore_axis_name="s",
                                num_cores=NC, num_subcores=NS),
    compiler_params=pltpu.CompilerParams(use_tc_tiling_on_sc=False,
                                         needs_layout_passes=False),
    scratch_types=[
        pltpu.VMEM((BPW, N_CAT), jnp.int32),
        pltpu.VMEM((BPW, N_NUM), jnp.float32),
        pltpu.VMEM((N_CAT, BPW), jnp.int32),
        pltpu.VMEM((BPW, D), jnp.float32),
        pltpu.VMEM((BPW, D), jnp.float32),
        pltpu.VMEM((D // 8, SPF, 8, 128), jnp.float32),
        pltpu.VMEM((D // 8, SPF, 8, 128), jnp.float32),
        pltpu.VMEM((N_NUM, D), jnp.float32),
        pltpu.VMEM((N_NUM, D), jnp.float32),
        pltpu.VMEM((N_CAT, D), jnp.float32),
        pltpu.SemaphoreType.DMA,
        pltpu.SemaphoreType.DMA,
        pltpu.SemaphoreType.DMA,
        pltpu.SemaphoreType.DMA,
    ],
)

_TBLK = 2048
_QK = 655360


def _relayout_body(q0_ref, q1_ref, q2_ref, q3_ref, dst_ref):
    xs = jnp.concatenate(
        [q0_ref[...], q1_ref[...], q2_ref[...], q3_ref[...]], axis=0)
    dst_ref[...] = jnp.transpose(xs)


_tbl_relayout = pl.pallas_call(
    _relayout_body,
    grid=(_QK // _TBLK,),
    in_specs=[pl.BlockSpec(
        (32, _TBLK),
        lambda m, q=q: (0, jnp.minimum(q * (_QK // _TBLK) + m,
                                       (2600000 + _TBLK - 1) // _TBLK - 1)))
              for q in range(4)],
    out_specs=pl.BlockSpec((_TBLK, 128), lambda m: (m, 0)),
    out_shape=jax.ShapeDtypeStruct((_QK, 128), jnp.float32),
)


@jax.jit
def kernel(x_num, x_cat, num_weight, num_bias, cat_table, cat_bias):
    tbl_t = jax.lax.optimization_barrier(cat_table.T)
    tbl_lin = jax.lax.optimization_barrier(
        _tbl_relayout(tbl_t, tbl_t, tbl_t, tbl_t))
    tbl2 = jnp.reshape(tbl_lin, (4 * _QK, 32))
    out5 = _sc_tokenize(jnp.asarray(x_num, jnp.float32),
                        jnp.asarray(x_cat, jnp.int32),
                        num_weight, num_bias, tbl2, cat_bias)
    return out5.transpose((2, 4, 0, 1, 3)).reshape(B, N_NUM + N_CAT, D)


# speedup vs baseline: 6.7131x; 1.4654x over previous
import jax
import jax.numpy as jnp
from jax import lax
from jax.experimental import pallas as pl
from jax.experimental.pallas import tpu as pltpu
from jax.experimental.pallas import tpu_sc as plsc

N_CAT = 26
N_NUM = 13
D = 32
B = 16384
CAT_SIZE = 100000

NC = 2
NS = 16
NW = NC * NS
BPW = B // NW
IDX_CHUNK = 128
SPF = BPW // IDX_CHUNK
L = 16
QK = 655360


def _body(xnum_hbm, xcat_hbm, nw_hbm, nb_hbm, tbl_hbm, cb_hbm, out_hbm,
          xcat_v, xnum_v, idxc_v, gbuf0, gbuf1, obuf0, obuf1,
          nw_v, nb_v, cb_v, gsem0, gsem1, osem0, osem1):
    wid = lax.axis_index("c") * NS + lax.axis_index("s")
    b0 = wid * BPW

    pltpu.sync_copy(xcat_hbm.at[pl.ds(b0, BPW), :], xcat_v)
    pltpu.sync_copy(xnum_hbm.at[pl.ds(b0, BPW), :], xnum_v)
    pltpu.sync_copy(nw_hbm, nw_v)
    pltpu.sync_copy(nb_hbm, nb_v)
    pltpu.sync_copy(cb_hbm, cb_v)

    lanes = lax.iota(jnp.int32, L)
    kd0 = lanes // 8
    sd0 = lanes % 8

    def build_col(f):
        col = jnp.full((L,), f, jnp.int32)
        off = jnp.full((L,), f * CAT_SIZE, jnp.int32)

        @pl.loop(0, BPW // L)
        def _(k, f=f, col=col, off=off):
            rows = k * L + lanes
            vals = plsc.load_gather(xcat_v, [rows, col]) + off
            q = ((vals >= QK).astype(jnp.int32)
                 + (vals >= 2 * QK).astype(jnp.int32)
                 + (vals >= 3 * QK).astype(jnp.int32))
            idxc_v[f, pl.ds(k * L, L)] = 4 * (vals - q * QK) + q

    gbuf = (gbuf0, gbuf1)
    obuf = (obuf0, obuf1)
    gsems = (gsem0, gsem1)
    osems = (osem0, osem1)

    def fire_gather(f):
        descs = []
        for c in range(SPF):
            descs.append(pltpu.async_copy(
                tbl_hbm.at[idxc_v.at[f, pl.ds(c * IDX_CHUNK, IDX_CHUNK)]],
                gbuf[f % 2].at[pl.ds(c * IDX_CHUNK, IDX_CHUNK), :],
                gsems[f % 2]))
        return descs

    build_col(0)
    g_descs = {0: fire_gather(0)}

    out_descs = {}
    for f in range(N_CAT):
        buf = f % 2
        if f + 1 < N_CAT:
            build_col(f + 1)
            g_descs[f + 1] = fire_gather(f + 1)
        for d in g_descs[f]:
            d.wait()
        if f - 2 >= 0:
            out_descs[f - 2].wait()
        g = gbuf[buf]
        o = obuf[buf]
        cb_lo = cb_v[f, pl.ds(0, L)]
        cb_hi = cb_v[f, pl.ds(L, L)]

        @pl.loop(0, BPW, unroll=2)
        def _bias_tr(b, g=g, o=o, cb_lo=cb_lo, cb_hi=cb_hi):
            mlv = jnp.zeros((L,), jnp.int32) + b // IDX_CHUNK
            cbv = jnp.zeros((L,), jnp.int32) + b % IDX_CHUNK
            v0 = g[b, pl.ds(0, L)] + cb_lo
            v1 = g[b, pl.ds(L, L)] + cb_hi
            plsc.store_scatter(o, [kd0, mlv, sd0, cbv], v0)
            plsc.store_scatter(o, [kd0 + 2, mlv, sd0, cbv], v1)

        out_descs[f] = pltpu.async_copy(
            o.at[:, :, :, pl.ds(0, 128)],
            out_hbm.at[N_NUM + f, :, pl.ds(4 * wid, SPF), :, :],
            osems[buf])

    num_descs = {}
    for j in range(N_NUM):
        buf = j % 2
        if j < 2:
            out_descs[N_CAT - 2 + j].wait()
        else:
            num_descs[j - 2].wait()
        o = obuf[buf]
        colj = jnp.full((L,), j, jnp.int32)
        w_lo = nw_v[j, pl.ds(0, L)]
        w_hi = nw_v[j, pl.ds(L, L)]
        a_lo = nb_v[j, pl.ds(0, L)]
        a_hi = nb_v[j, pl.ds(L, L)]

        @pl.loop(0, BPW // L)
        def _num_emb(i, o=o, colj=colj, w_lo=w_lo, w_hi=w_hi,
                     a_lo=a_lo, a_hi=a_hi):
            ml = i // (IDX_CHUNK // L)
            gg = i % (IDX_CHUNK // L)
            rows = i * L + lanes
            xv = plsc.load_gather(xnum_v, [rows, colj])
            for d in range(D):
                w_d = w_lo[d] if d < L else w_hi[d - L]
                a_d = a_lo[d] if d < L else a_hi[d - L]
                o[d // 8, ml, d % 8, pl.ds(gg * L, L)] = xv * w_d + a_d

        num_descs[j] = pltpu.async_copy(
            o.at[:, :, :, pl.ds(0, 128)],
            out_hbm.at[j, :, pl.ds(4 * wid, SPF), :, :], osems[buf])

    num_descs[N_NUM - 2].wait()
    num_descs[N_NUM - 1].wait()


_sc_tokenize = pl.kernel(
    _body,
    out_type=jax.ShapeDtypeStruct((N_NUM + N_CAT, D // 8, B // 128, 8, 128),
                                  jnp.float32),
    mesh=plsc.VectorSubcoreMesh(core_axis_name="c", subcore_axis_name="s",
                                num_cores=NC, num_subcores=NS),
    compiler_params=pltpu.CompilerParams(use_tc_tiling_on_sc=False,
                                         needs_layout_passes=False),
    scratch_types=[
        pltpu.VMEM((BPW, N_CAT), jnp.int32),
        pltpu.VMEM((BPW, N_NUM), jnp.float32),
        pltpu.VMEM((N_CAT, BPW), jnp.int32),
        pltpu.VMEM((BPW, D), jnp.float32),
        pltpu.VMEM((BPW, D), jnp.float32),
        pltpu.VMEM((D // 8, SPF, 8, 131), jnp.float32),
        pltpu.VMEM((D // 8, SPF, 8, 131), jnp.float32),
        pltpu.VMEM((N_NUM, D), jnp.float32),
        pltpu.VMEM((N_NUM, D), jnp.float32),
        pltpu.VMEM((N_CAT, D), jnp.float32),
        pltpu.SemaphoreType.DMA,
        pltpu.SemaphoreType.DMA,
        pltpu.SemaphoreType.DMA,
        pltpu.SemaphoreType.DMA,
    ],
)

_TBLK = 2048
_QK = 655360


def _relayout_body(q0_ref, q1_ref, q2_ref, q3_ref, dst_ref):
    xs = jnp.concatenate(
        [q0_ref[...], q1_ref[...], q2_ref[...], q3_ref[...]], axis=0)
    dst_ref[...] = jnp.transpose(xs)


_tbl_relayout = pl.pallas_call(
    _relayout_body,
    grid=(_QK // _TBLK,),
    in_specs=[pl.BlockSpec(
        (32, _TBLK),
        lambda m, q=q: (0, jnp.minimum(q * (_QK // _TBLK) + m,
                                       (2600000 + _TBLK - 1) // _TBLK - 1)))
              for q in range(4)],
    out_specs=pl.BlockSpec((_TBLK, 128), lambda m: (m, 0)),
    out_shape=jax.ShapeDtypeStruct((_QK, 128), jnp.float32),
)


@jax.jit
def kernel(x_num, x_cat, num_weight, num_bias, cat_table, cat_bias):
    tbl_t = jax.lax.optimization_barrier(cat_table.T)
    tbl_lin = jax.lax.optimization_barrier(
        _tbl_relayout(tbl_t, tbl_t, tbl_t, tbl_t))
    tbl2 = jnp.reshape(tbl_lin, (4 * _QK, 32))
    out5 = _sc_tokenize(jnp.asarray(x_num, jnp.float32),
                        jnp.asarray(x_cat, jnp.int32),
                        num_weight, num_bias, tbl2, cat_bias)
    return out5.transpose((2, 4, 0, 1, 3)).reshape(B, N_NUM + N_CAT, D)


# speedup vs baseline: 8.1562x; 1.2150x over previous
import jax
import jax.numpy as jnp
from jax import lax
from jax.experimental import pallas as pl
from jax.experimental.pallas import tpu as pltpu
from jax.experimental.pallas import tpu_sc as plsc

N_CAT = 26
N_NUM = 13
D = 32
B = 16384
CAT_SIZE = 100000

NC = 2
NS = 16
NW = NC * NS
BPW = B // NW
IDX_CHUNK = 128
SPF = BPW // IDX_CHUNK
L = 16
QK = 655360


def _body(xnum_hbm, xcat_hbm, nw_hbm, nb_hbm, tbl_hbm, cb_hbm, out_hbm,
          xcat_v, xnum_v, idxc_v, gbuf0, gbuf1, obuf0, obuf1,
          nw_v, nb_v, cb_v, gsem0, gsem1, osem0, osem1):
    wid = lax.axis_index("c") * NS + lax.axis_index("s")
    b0 = wid * BPW

    pltpu.sync_copy(xcat_hbm.at[pl.ds(b0, BPW), :], xcat_v)
    pltpu.sync_copy(xnum_hbm.at[pl.ds(b0, BPW), :], xnum_v)
    pltpu.sync_copy(nw_hbm, nw_v)
    pltpu.sync_copy(nb_hbm, nb_v)
    pltpu.sync_copy(cb_hbm, cb_v)

    lanes = lax.iota(jnp.int32, L)
    kd0 = lanes // 8
    sd0 = lanes % 8

    def build_col(f):
        col = jnp.full((L,), f, jnp.int32)
        off = jnp.full((L,), f * CAT_SIZE, jnp.int32)

        @pl.loop(0, BPW // L)
        def _(k, f=f, col=col, off=off):
            rows = k * L + lanes
            vals = plsc.load_gather(xcat_v, [rows, col]) + off
            q = ((vals >= QK).astype(jnp.int32)
                 + (vals >= 2 * QK).astype(jnp.int32)
                 + (vals >= 3 * QK).astype(jnp.int32))
            idxc_v[f, pl.ds(k * L, L)] = 4 * (vals - q * QK) + q

    gbuf = (gbuf0, gbuf1)
    obuf = (obuf0, obuf1)
    gsems = (gsem0, gsem1)
    osems = (osem0, osem1)

    def fire_gather(f):
        descs = []
        for c in range(SPF):
            descs.append(pltpu.async_copy(
                tbl_hbm.at[idxc_v.at[f, pl.ds(c * IDX_CHUNK, IDX_CHUNK)]],
                gbuf[f % 2].at[pl.ds(c * IDX_CHUNK, IDX_CHUNK), :],
                gsems[f % 2]))
        return descs

    build_col(0)
    g_descs = {0: fire_gather(0)}

    out_descs = {}
    for f in range(N_CAT):
        buf = f % 2
        if f + 1 < N_CAT:
            build_col(f + 1)
            g_descs[f + 1] = fire_gather(f + 1)
        for d in g_descs[f]:
            d.wait()
        if f - 2 >= 0:
            out_descs[f - 2].wait()
        g = gbuf[buf]
        o = obuf[buf]
        cb_lo = cb_v[f, pl.ds(0, L)]
        cb_hi = cb_v[f, pl.ds(L, L)]

        @pl.loop(0, BPW, unroll=2)
        def _bias_tr(b, g=g, o=o, cb_lo=cb_lo, cb_hi=cb_hi):
            mlv = jnp.zeros((L,), jnp.int32) + b // IDX_CHUNK
            cbv = jnp.zeros((L,), jnp.int32) + b % IDX_CHUNK
            v0 = g[b, pl.ds(0, L)] + cb_lo
            v1 = g[b, pl.ds(L, L)] + cb_hi
            plsc.store_scatter(o, [kd0, mlv, sd0, cbv], v0)
            plsc.store_scatter(o, [kd0 + 2, mlv, sd0, cbv], v1)

        out_descs[f] = pltpu.async_copy(
            o.at[:, :, :, pl.ds(0, 128)],
            out_hbm.at[N_NUM + f, :, pl.ds(4 * wid, SPF), :, :],
            osems[buf])

    num_descs = {}
    for j in range(N_NUM):
        buf = j % 2
        if j < 2:
            out_descs[N_CAT - 2 + j].wait()
        else:
            num_descs[j - 2].wait()
        o = obuf[buf]
        colj = jnp.full((L,), j, jnp.int32)
        w_lo = nw_v[j, pl.ds(0, L)]
        w_hi = nw_v[j, pl.ds(L, L)]
        a_lo = nb_v[j, pl.ds(0, L)]
        a_hi = nb_v[j, pl.ds(L, L)]

        @pl.loop(0, BPW // L)
        def _num_emb(i, o=o, colj=colj, w_lo=w_lo, w_hi=w_hi,
                     a_lo=a_lo, a_hi=a_hi):
            ml = i // (IDX_CHUNK // L)
            gg = i % (IDX_CHUNK // L)
            rows = i * L + lanes
            xv = plsc.load_gather(xnum_v, [rows, colj])
            for d in range(D):
                w_d = w_lo[d] if d < L else w_hi[d - L]
                a_d = a_lo[d] if d < L else a_hi[d - L]
                o[d // 8, ml, d % 8, pl.ds(gg * L, L)] = xv * w_d + a_d

        num_descs[j] = pltpu.async_copy(
            o.at[:, :, :, pl.ds(0, 128)],
            out_hbm.at[j, :, pl.ds(4 * wid, SPF), :, :], osems[buf])

    num_descs[N_NUM - 2].wait()
    num_descs[N_NUM - 1].wait()


_sc_tokenize = pl.kernel(
    _body,
    out_type=jax.ShapeDtypeStruct((N_NUM + N_CAT, D // 8, B // 128, 8, 128),
                                  jnp.float32),
    mesh=plsc.VectorSubcoreMesh(core_axis_name="c", subcore_axis_name="s",
                                num_cores=NC, num_subcores=NS),
    compiler_params=pltpu.CompilerParams(use_tc_tiling_on_sc=False,
                                         needs_layout_passes=False),
    scratch_types=[
        pltpu.VMEM((BPW, N_CAT), jnp.int32),
        pltpu.VMEM((BPW, N_NUM), jnp.float32),
        pltpu.VMEM((N_CAT, BPW), jnp.int32),
        pltpu.VMEM((BPW, D), jnp.float32),
        pltpu.VMEM((BPW, D), jnp.float32),
        pltpu.VMEM((D // 8, SPF, 8, 131), jnp.float32),
        pltpu.VMEM((D // 8, SPF, 8, 131), jnp.float32),
        pltpu.VMEM((N_NUM, D), jnp.float32),
        pltpu.VMEM((N_NUM, D), jnp.float32),
        pltpu.VMEM((N_CAT, D), jnp.float32),
        pltpu.SemaphoreType.DMA,
        pltpu.SemaphoreType.DMA,
        pltpu.SemaphoreType.DMA,
        pltpu.SemaphoreType.DMA,
    ],
)

_TBLK = 4096
_QK = 655360


def _relayout_body(q0_ref, q1_ref, q2_ref, q3_ref, dst_ref):
    xs = jnp.concatenate(
        [q0_ref[...], q1_ref[...], q2_ref[...], q3_ref[...]], axis=0)
    dst_ref[...] = jnp.transpose(xs)


_tbl_relayout = pl.pallas_call(
    _relayout_body,
    grid=(_QK // _TBLK,),
    in_specs=[pl.BlockSpec(
        (32, _TBLK),
        lambda m, q=q: (0, jnp.minimum(q * (_QK // _TBLK) + m,
                                       (2600000 + _TBLK - 1) // _TBLK - 1)))
              for q in range(4)],
    out_specs=pl.BlockSpec((_TBLK, 128), lambda m: (m, 0)),
    out_shape=jax.ShapeDtypeStruct((_QK, 128), jnp.float32),
)


@jax.jit
def kernel(x_num, x_cat, num_weight, num_bias, cat_table, cat_bias):
    tbl_t = jax.lax.optimization_barrier(cat_table.T)
    tbl_lin = jax.lax.optimization_barrier(
        _tbl_relayout(tbl_t, tbl_t, tbl_t, tbl_t))
    tbl2 = jnp.reshape(tbl_lin, (4 * _QK, 32))
    out5 = _sc_tokenize(jnp.asarray(x_num, jnp.float32),
                        jnp.asarray(x_cat, jnp.int32),
                        num_weight, num_bias, tbl2, cat_bias)
    return out5.transpose((2, 4, 0, 1, 3)).reshape(B, N_NUM + N_CAT, D)


# speedup vs baseline: 8.8974x; 1.0909x over previous
import jax
import jax.numpy as jnp
from jax import lax
from jax.experimental import pallas as pl
from jax.experimental.pallas import tpu as pltpu
from jax.experimental.pallas import tpu_sc as plsc

N_CAT = 26
N_NUM = 13
D = 32
B = 16384
CAT_SIZE = 100000

NC = 2
NS = 16
NW = NC * NS
BPW = B // NW
IDX_CHUNK = 128
SPF = BPW // IDX_CHUNK
L = 16
QK = 655360


def _body(xnum_hbm, xcat_hbm, nw_hbm, nb_hbm, tbl_hbm, cb_hbm, out_hbm,
          xcat_v, xnum_v, idxc_v, gbuf0, gbuf1, obuf0, obuf1,
          nw_v, nb_v, cb_v, gsem0, gsem1, osem0, osem1):
    wid = lax.axis_index("c") * NS + lax.axis_index("s")
    b0 = wid * BPW

    pltpu.sync_copy(xcat_hbm.at[pl.ds(b0, BPW), :], xcat_v)
    pltpu.sync_copy(xnum_hbm.at[pl.ds(b0, BPW), :], xnum_v)
    pltpu.sync_copy(nw_hbm, nw_v)
    pltpu.sync_copy(nb_hbm, nb_v)
    pltpu.sync_copy(cb_hbm, cb_v)

    lanes = lax.iota(jnp.int32, L)
    kd0 = lanes // 8
    sd0 = lanes % 8

    def build_col(f):
        col = jnp.full((L,), f, jnp.int32)
        off = jnp.full((L,), f * CAT_SIZE, jnp.int32)

        @pl.loop(0, BPW // L)
        def _(k, f=f, col=col, off=off):
            rows = k * L + lanes
            vals = plsc.load_gather(xcat_v, [rows, col]) + off
            q = ((vals >= QK).astype(jnp.int32)
                 + (vals >= 2 * QK).astype(jnp.int32)
                 + (vals >= 3 * QK).astype(jnp.int32))
            idxc_v[f, pl.ds(k * L, L)] = 4 * (vals - q * QK) + q

    gbuf = (gbuf0, gbuf1)
    obuf = (obuf0, obuf1)
    gsems = (gsem0, gsem1)
    osems = (osem0, osem1)

    def fire_gather(f):
        descs = []
        for c in range(SPF):
            descs.append(pltpu.async_copy(
                tbl_hbm.at[idxc_v.at[f, pl.ds(c * IDX_CHUNK, IDX_CHUNK)]],
                gbuf[f % 2].at[pl.ds(c * IDX_CHUNK, IDX_CHUNK), :],
                gsems[f % 2]))
        return descs

    build_col(0)
    g_descs = {0: fire_gather(0)}

    out_descs = {}
    for f in range(N_CAT):
        buf = f % 2
        if f + 1 < N_CAT:
            build_col(f + 1)
            g_descs[f + 1] = fire_gather(f + 1)
        for d in g_descs[f]:
            d.wait()
        if f - 2 >= 0:
            out_descs[f - 2].wait()
        g = gbuf[buf]
        o = obuf[buf]
        cb_lo = cb_v[f, pl.ds(0, L)]
        cb_hi = cb_v[f, pl.ds(L, L)]

        @pl.loop(0, BPW, unroll=2)
        def _bias_tr(b, g=g, o=o, cb_lo=cb_lo, cb_hi=cb_hi):
            mlv = jnp.zeros((L,), jnp.int32) + b // IDX_CHUNK
            cbv = jnp.zeros((L,), jnp.int32) + b % IDX_CHUNK
            v0 = g[b, pl.ds(0, L)] + cb_lo
            v1 = g[b, pl.ds(L, L)] + cb_hi
            plsc.store_scatter(o, [kd0, mlv, sd0, cbv], v0)
            plsc.store_scatter(o, [kd0 + 2, mlv, sd0, cbv], v1)

        out_descs[f] = pltpu.async_copy(
            o.at[:, :, :, pl.ds(0, 128)],
            out_hbm.at[N_NUM + f, :, pl.ds(4 * wid, SPF), :, :],
            osems[buf])

    num_descs = {}
    for j in range(N_NUM):
        buf = j % 2
        if j < 2:
            out_descs[N_CAT - 2 + j].wait()
        else:
            num_descs[j - 2].wait()
        o = obuf[buf]
        colj = jnp.full((L,), j, jnp.int32)
        w_lo = nw_v[j, pl.ds(0, L)]
        w_hi = nw_v[j, pl.ds(L, L)]
        a_lo = nb_v[j, pl.ds(0, L)]
        a_hi = nb_v[j, pl.ds(L, L)]

        @pl.loop(0, BPW // L)
        def _num_emb(i, o=o, colj=colj, w_lo=w_lo, w_hi=w_hi,
                     a_lo=a_lo, a_hi=a_hi):
            ml = i // (IDX_CHUNK // L)
            gg = i % (IDX_CHUNK // L)
            rows = i * L + lanes
            xv = plsc.load_gather(xnum_v, [rows, colj])
            for d in range(D):
                w_d = w_lo[d] if d < L else w_hi[d - L]
                a_d = a_lo[d] if d < L else a_hi[d - L]
                o[d // 8, ml, d % 8, pl.ds(gg * L, L)] = xv * w_d + a_d

        num_descs[j] = pltpu.async_copy(
            o.at[:, :, :, pl.ds(0, 128)],
            out_hbm.at[j, :, pl.ds(4 * wid, SPF), :, :], osems[buf])

    num_descs[N_NUM - 2].wait()
    num_descs[N_NUM - 1].wait()


_sc_tokenize = pl.kernel(
    _body,
    out_type=jax.ShapeDtypeStruct((N_NUM + N_CAT, D // 8, B // 128, 8, 128),
                                  jnp.float32),
    mesh=plsc.VectorSubcoreMesh(core_axis_name="c", subcore_axis_name="s",
                                num_cores=NC, num_subcores=NS),
    compiler_params=pltpu.CompilerParams(use_tc_tiling_on_sc=False,
                                         needs_layout_passes=False),
    scratch_types=[
        pltpu.VMEM((BPW, N_CAT), jnp.int32),
        pltpu.VMEM((BPW, N_NUM), jnp.float32),
        pltpu.VMEM((N_CAT, BPW), jnp.int32),
        pltpu.VMEM((BPW, D), jnp.float32),
        pltpu.VMEM((BPW, D), jnp.float32),
        pltpu.VMEM((D // 8, SPF, 8, 131), jnp.float32),
        pltpu.VMEM((D // 8, SPF, 8, 131), jnp.float32),
        pltpu.VMEM((N_NUM, D), jnp.float32),
        pltpu.VMEM((N_NUM, D), jnp.float32),
        pltpu.VMEM((N_CAT, D), jnp.float32),
        pltpu.SemaphoreType.DMA,
        pltpu.SemaphoreType.DMA,
        pltpu.SemaphoreType.DMA,
        pltpu.SemaphoreType.DMA,
    ],
)

_TBLK = 8192
_QK = 655360


def _relayout_body(q0_ref, q1_ref, q2_ref, q3_ref, dst_ref):
    xs = jnp.concatenate(
        [q0_ref[...], q1_ref[...], q2_ref[...], q3_ref[...]], axis=0)
    dst_ref[...] = jnp.transpose(xs)


_tbl_relayout = pl.pallas_call(
    _relayout_body,
    grid=(_QK // _TBLK,),
    in_specs=[pl.BlockSpec(
        (32, _TBLK),
        lambda m, q=q: (0, jnp.minimum(q * (_QK // _TBLK) + m,
                                       (2600000 + _TBLK - 1) // _TBLK - 1)))
              for q in range(4)],
    out_specs=pl.BlockSpec((_TBLK, 128), lambda m: (m, 0)),
    out_shape=jax.ShapeDtypeStruct((_QK, 128), jnp.float32),
)


@jax.jit
def kernel(x_num, x_cat, num_weight, num_bias, cat_table, cat_bias):
    tbl_t = jax.lax.optimization_barrier(cat_table.T)
    tbl_lin = jax.lax.optimization_barrier(
        _tbl_relayout(tbl_t, tbl_t, tbl_t, tbl_t))
    tbl2 = jnp.reshape(tbl_lin, (4 * _QK, 32))
    out5 = _sc_tokenize(jnp.asarray(x_num, jnp.float32),
                        jnp.asarray(x_cat, jnp.int32),
                        num_weight, num_bias, tbl2, cat_bias)
    return out5.transpose((2, 4, 0, 1, 3)).reshape(B, N_NUM + N_CAT, D)


# speedup vs baseline: 9.0202x; 1.0138x over previous
import jax
import jax.numpy as jnp
from jax import lax
from jax.experimental import pallas as pl
from jax.experimental.pallas import tpu as pltpu
from jax.experimental.pallas import tpu_sc as plsc

N_CAT = 26
N_NUM = 13
D = 32
B = 16384
CAT_SIZE = 100000

NC = 2
NS = 16
NW = NC * NS
BPW = B // NW
IDX_CHUNK = 128
SPF = BPW // IDX_CHUNK
L = 16
QK = 655360


def _body(xnum_hbm, xcat_hbm, nw_hbm, nb_hbm, tbl_hbm, cb_hbm, out_hbm,
          xcat_v, xnum_v, idxc_v, gbuf0, gbuf1, obuf0, obuf1,
          nw_v, nb_v, cb_v, gsem0, gsem1, osem0, osem1):
    wid = lax.axis_index("c") * NS + lax.axis_index("s")
    b0 = wid * BPW

    pltpu.sync_copy(xcat_hbm.at[pl.ds(b0, BPW), :], xcat_v)
    pltpu.sync_copy(xnum_hbm.at[pl.ds(b0, BPW), :], xnum_v)
    pltpu.sync_copy(nw_hbm, nw_v)
    pltpu.sync_copy(nb_hbm, nb_v)
    pltpu.sync_copy(cb_hbm, cb_v)

    lanes = lax.iota(jnp.int32, L)
    kd0 = lanes // 8
    sd0 = lanes % 8

    def build_col(f):
        col = jnp.full((L,), f, jnp.int32)
        off = jnp.full((L,), f * CAT_SIZE, jnp.int32)

        @pl.loop(0, BPW // L)
        def _(k, f=f, col=col, off=off):
            rows = k * L + lanes
            vals = plsc.load_gather(xcat_v, [rows, col]) + off
            q = ((vals >= QK).astype(jnp.int32)
                 + (vals >= 2 * QK).astype(jnp.int32)
                 + (vals >= 3 * QK).astype(jnp.int32))
            idxc_v[f, pl.ds(k * L, L)] = 4 * (vals - q * QK) + q

    gbuf = (gbuf0, gbuf1)
    obuf = (obuf0, obuf1)
    gsems = (gsem0, gsem1)
    osems = (osem0, osem1)

    def fire_gather(f):
        descs = []
        for c in range(SPF):
            descs.append(pltpu.async_copy(
                tbl_hbm.at[idxc_v.at[f, pl.ds(c * IDX_CHUNK, IDX_CHUNK)]],
                gbuf[f % 2].at[pl.ds(c * IDX_CHUNK, IDX_CHUNK), :],
                gsems[f % 2]))
        return descs

    build_col(0)
    g_descs = {0: fire_gather(0)}

    out_descs = {}
    for f in range(N_CAT):
        buf = f % 2
        if f + 1 < N_CAT:
            build_col(f + 1)
            g_descs[f + 1] = fire_gather(f + 1)
        for d in g_descs[f]:
            d.wait()
        if f - 2 >= 0:
            out_descs[f - 2].wait()
        g = gbuf[buf]
        o = obuf[buf]
        cb_lo = cb_v[f, pl.ds(0, L)]
        cb_hi = cb_v[f, pl.ds(L, L)]

        @pl.loop(0, BPW, unroll=2)
        def _bias_tr(b, g=g, o=o, cb_lo=cb_lo, cb_hi=cb_hi):
            mlv = jnp.zeros((L,), jnp.int32) + b // IDX_CHUNK
            cbv = jnp.zeros((L,), jnp.int32) + b % IDX_CHUNK
            v0 = g[b, pl.ds(0, L)] + cb_lo
            v1 = g[b, pl.ds(L, L)] + cb_hi
            plsc.store_scatter(o, [kd0, mlv, sd0, cbv], v0)
            plsc.store_scatter(o, [kd0 + 2, mlv, sd0, cbv], v1)

        out_descs[f] = pltpu.async_copy(
            o.at[:, :, :, pl.ds(0, 128)],
            out_hbm.at[N_NUM + f, :, pl.ds(4 * wid, SPF), :, :],
            osems[buf])

    num_descs = {}
    for j in range(N_NUM):
        buf = j % 2
        if j < 2:
            out_descs[N_CAT - 2 + j].wait()
        else:
            num_descs[j - 2].wait()
        o = obuf[buf]
        colj = jnp.full((L,), j, jnp.int32)
        w_lo = nw_v[j, pl.ds(0, L)]
        w_hi = nw_v[j, pl.ds(L, L)]
        a_lo = nb_v[j, pl.ds(0, L)]
        a_hi = nb_v[j, pl.ds(L, L)]

        @pl.loop(0, BPW // L)
        def _num_emb(i, o=o, colj=colj, w_lo=w_lo, w_hi=w_hi,
                     a_lo=a_lo, a_hi=a_hi):
            ml = i // (IDX_CHUNK // L)
            gg = i % (IDX_CHUNK // L)
            rows = i * L + lanes
            xv = plsc.load_gather(xnum_v, [rows, colj])
            for d in range(D):
                w_d = w_lo[d] if d < L else w_hi[d - L]
                a_d = a_lo[d] if d < L else a_hi[d - L]
                o[d // 8, ml, d % 8, pl.ds(gg * L, L)] = xv * w_d + a_d

        num_descs[j] = pltpu.async_copy(
            o.at[:, :, :, pl.ds(0, 128)],
            out_hbm.at[j, :, pl.ds(4 * wid, SPF), :, :], osems[buf])

    num_descs[N_NUM - 2].wait()
    num_descs[N_NUM - 1].wait()


_sc_tokenize = pl.kernel(
    _body,
    out_type=jax.ShapeDtypeStruct((N_NUM + N_CAT, D // 8, B // 128, 8, 128),
                                  jnp.float32),
    mesh=plsc.VectorSubcoreMesh(core_axis_name="c", subcore_axis_name="s",
                                num_cores=NC, num_subcores=NS),
    compiler_params=pltpu.CompilerParams(use_tc_tiling_on_sc=False,
                                         needs_layout_passes=False),
    scratch_types=[
        pltpu.VMEM((BPW, N_CAT), jnp.int32),
        pltpu.VMEM((BPW, N_NUM), jnp.float32),
        pltpu.VMEM((N_CAT, BPW), jnp.int32),
        pltpu.VMEM((BPW, D), jnp.float32),
        pltpu.VMEM((BPW, D), jnp.float32),
        pltpu.VMEM((D // 8, SPF, 8, 131), jnp.float32),
        pltpu.VMEM((D // 8, SPF, 8, 131), jnp.float32),
        pltpu.VMEM((N_NUM, D), jnp.float32),
        pltpu.VMEM((N_NUM, D), jnp.float32),
        pltpu.VMEM((N_CAT, D), jnp.float32),
        pltpu.SemaphoreType.DMA,
        pltpu.SemaphoreType.DMA,
        pltpu.SemaphoreType.DMA,
        pltpu.SemaphoreType.DMA,
    ],
)

_TBLK = 16384
_QK = 655360


def _relayout_body(q0_ref, q1_ref, q2_ref, q3_ref, dst_ref):
    xs = jnp.concatenate(
        [q0_ref[...], q1_ref[...], q2_ref[...], q3_ref[...]], axis=0)
    dst_ref[...] = jnp.transpose(xs)


_tbl_relayout = pl.pallas_call(
    _relayout_body,
    grid=(_QK // _TBLK,),
    in_specs=[pl.BlockSpec(
        (32, _TBLK),
        lambda m, q=q: (0, jnp.minimum(q * (_QK // _TBLK) + m,
                                       (2600000 + _TBLK - 1) // _TBLK - 1)))
              for q in range(4)],
    out_specs=pl.BlockSpec((_TBLK, 128), lambda m: (m, 0)),
    out_shape=jax.ShapeDtypeStruct((_QK, 128), jnp.float32),
)


@jax.jit
def kernel(x_num, x_cat, num_weight, num_bias, cat_table, cat_bias):
    tbl_t = jax.lax.optimization_barrier(cat_table.T)
    tbl_lin = jax.lax.optimization_barrier(
        _tbl_relayout(tbl_t, tbl_t, tbl_t, tbl_t))
    tbl2 = jnp.reshape(tbl_lin, (4 * _QK, 32))
    out5 = _sc_tokenize(jnp.asarray(x_num, jnp.float32),
                        jnp.asarray(x_cat, jnp.int32),
                        num_weight, num_bias, tbl2, cat_bias)
    return out5.transpose((2, 4, 0, 1, 3)).reshape(B, N_NUM + N_CAT, D)
